# baseline (speedup 1.0000x reference)
import functools

import jax
import jax.numpy as jnp
from jax import lax
from jax.experimental import pallas as pl
from jax.experimental.pallas import tpu as pltpu
from jax.experimental.pallas import tpu_sc as plsc

N = 10000
E = 320000
NW = 32
NC = 2
NS = 16
EPW = E // NW
K = 80
NCHUNK = EPW // K
SEG = 25
NSEG = NCHUNK // SEG
NA = N
RPT = 624
RPT_LAST = N - (NS - 1) * RPT

_f32 = jnp.float32


def _rowpart(s, fn):
    @pl.when(s < NS - 1)
    def _():
        fn(pl.multiple_of(s * RPT, 8), RPT)

    @pl.when(s == NS - 1)
    def _():
        fn((NS - 1) * RPT, RPT_LAST)


def _sc_mesh():
    return plsc.VectorSubcoreMesh(core_axis_name="c", subcore_axis_name="s",
                                  num_cores=NC, num_subcores=NS)


@functools.lru_cache(maxsize=None)
def _make_deg_kernel():
    @functools.partial(
        pl.kernel,
        mesh=_sc_mesh(),
        out_type=jax.ShapeDtypeStruct((NC, N, 128), _f32),
        scratch_types=[
            pltpu.VMEM((SEG, K), jnp.int32),
            pltpu.VMEM((K, 128), _f32),
            pltpu.SemaphoreType.DMA,
            pltpu.VMEM_SHARED((NA, 128), _f32),
        ],
    )
    def deg_kernel(dst_hbm, ones_hbm, zeros_hbm, out_hbm, dst_v, ones_v, sem,
                   acc):
        c = lax.axis_index("c")
        s = lax.axis_index("s")
        wid = s * NC + c

        pltpu.sync_copy(ones_hbm, ones_v)
        _rowpart(s, lambda off, sz: pltpu.sync_copy(
            zeros_hbm.at[pl.ds(0, sz)], acc.at[pl.ds(off, sz)]))
        plsc.subcore_barrier()

        def seg_body(g, carry):
            pltpu.sync_copy(dst_hbm.at[wid * NSEG + g], dst_v)

            def fire(j, c2):
                pltpu.async_copy(ones_v, acc.at[dst_v.at[j]], sem, add=True)
                return c2

            lax.fori_loop(0, SEG, fire, 0)

            def drain(j, c2):
                pltpu.make_async_copy(ones_v, acc.at[dst_v.at[j]], sem).wait()
                return c2

            lax.fori_loop(0, SEG, drain, 0)
            return carry

        lax.fori_loop(0, NSEG, seg_body, 0)
        plsc.subcore_barrier()
        _rowpart(s, lambda off, sz: pltpu.sync_copy(
            acc.at[pl.ds(off, sz)], out_hbm.at[c].at[pl.ds(off, sz)]))

    return deg_kernel


@functools.lru_cache(maxsize=None)
def _make_agg_kernel(D):
    @functools.partial(
        pl.kernel,
        mesh=_sc_mesh(),
        out_type=jax.ShapeDtypeStruct((NC, N, D), _f32),
        scratch_types=[
            pltpu.VMEM((SEG * K,), jnp.int32),
            pltpu.VMEM((SEG, K), jnp.int32),
            pltpu.VMEM((K, D), _f32),
            pltpu.VMEM((K, D), _f32),
            pltpu.SemaphoreType.DMA,
            pltpu.SemaphoreType.DMA,
            pltpu.VMEM_SHARED((NA, D), _f32),
        ],
    )
    def agg_kernel(xw_hbm, src_hbm, dst_hbm, zeros_hbm, out_hbm,
                   src_v, dst_v, rows_a, rows_b, sem_a, sem_b, acc):
        c = lax.axis_index("c")
        s = lax.axis_index("s")
        wid = s * NC + c

        _rowpart(s, lambda off, sz: pltpu.sync_copy(
            zeros_hbm.at[pl.ds(0, sz)], acc.at[pl.ds(off, sz)]))
        plsc.subcore_barrier()

        def sidx(j):
            return src_v.at[pl.ds(j * K, K)]

        def seg_body(g, carry):
            pltpu.sync_copy(
                src_hbm.at[pl.ds((wid * NSEG + g) * SEG * K, SEG * K)], src_v)
            pltpu.sync_copy(dst_hbm.at[wid * NSEG + g], dst_v)
            pltpu.async_copy(xw_hbm.at[sidx(0)], rows_a, sem_a)

            def body(i, c2):
                j = 2 * i
                pltpu.async_copy(xw_hbm.at[sidx(j + 1)], rows_b, sem_b)
                pltpu.make_async_copy(
                    xw_hbm.at[sidx(j)], rows_a, sem_a).wait()
                pltpu.sync_copy(rows_a, acc.at[dst_v.at[j]], add=True)
                pltpu.async_copy(xw_hbm.at[sidx(j + 2)], rows_a, sem_a)
                pltpu.make_async_copy(
                    xw_hbm.at[sidx(j + 1)], rows_b, sem_b).wait()
                pltpu.sync_copy(rows_b, acc.at[dst_v.at[j + 1]], add=True)
                return c2

            lax.fori_loop(0, (SEG - 1) // 2, body, 0)
            pltpu.make_async_copy(xw_hbm.at[sidx(SEG - 1)], rows_a,
                                  sem_a).wait()
            pltpu.sync_copy(rows_a, acc.at[dst_v.at[SEG - 1]], add=True)
            return carry

        lax.fori_loop(0, NSEG, seg_body, 0)
        plsc.subcore_barrier()
        _rowpart(s, lambda off, sz: pltpu.sync_copy(
            acc.at[pl.ds(off, sz)], out_hbm.at[c].at[pl.ds(off, sz)]))

    return agg_kernel


_BR = 1000


def _mm1_body(deg_ref, x_ref, w_ref, xws_ref, dinv_ref):
    deg = deg_ref[0, :, 0:1] + deg_ref[1, :, 0:1] + 1.0
    dinv = lax.rsqrt(deg)
    xw = jnp.dot(x_ref[...], w_ref[...], preferred_element_type=_f32)
    xws_ref[...] = xw * dinv
    dinv_ref[...] = dinv


def _mm1(degp, X, W1):
    grid = N // _BR
    return pl.pallas_call(
        _mm1_body,
        grid=(grid,),
        in_specs=[
            pl.BlockSpec((NC, _BR, 128), lambda i: (0, i, 0)),
            pl.BlockSpec((_BR, 128), lambda i: (i, 0)),
            pl.BlockSpec((128, 128), lambda i: (0, 0)),
        ],
        out_specs=[
            pl.BlockSpec((_BR, 128), lambda i: (i, 0)),
            pl.BlockSpec((_BR, 1), lambda i: (i, 0)),
        ],
        out_shape=[
            jax.ShapeDtypeStruct((N, 128), _f32),
            jax.ShapeDtypeStruct((N, 1), _f32),
        ],
    )(degp, X, W1)


def _mm2_body(p_ref, xws_ref, dinv_ref, b1_ref, w2_ref, out_ref):
    dinv = dinv_ref[...]
    a = (p_ref[0] + p_ref[1] + xws_ref[...]) * dinv + b1_ref[...]
    a = jnp.maximum(a, 0.0)
    out_ref[...] = jnp.dot(a, w2_ref[...], preferred_element_type=_f32) * dinv


def _mm2(p, xws1, dinv, b1, W2):
    grid = N // _BR
    return pl.pallas_call(
        _mm2_body,
        grid=(grid,),
        in_specs=[
            pl.BlockSpec((NC, _BR, 128), lambda i: (0, i, 0)),
            pl.BlockSpec((_BR, 128), lambda i: (i, 0)),
            pl.BlockSpec((_BR, 1), lambda i: (i, 0)),
            pl.BlockSpec((128,), lambda i: (0,)),
            pl.BlockSpec((128, 128), lambda i: (0, 0)),
        ],
        out_specs=pl.BlockSpec((_BR, 128), lambda i: (i, 0)),
        out_shape=jax.ShapeDtypeStruct((N, 128), _f32),
    )(p, xws1, dinv, b1, W2)


def _mm3_body(q_ref, xws_ref, dinv_ref, b2_ref, wfc_ref, bfc_ref, z_ref):
    dinv = dinv_ref[...]
    h = (q_ref[0] + q_ref[1] + xws_ref[...]) * dinv + b2_ref[...]
    z = jnp.dot(h, wfc_ref[...], preferred_element_type=_f32) + bfc_ref[...]
    z_ref[...] = jnp.maximum(z, 0.0)


def _mm3(q, xws2, dinv, b2, Wfc, bfc):
    grid = N // _BR
    return pl.pallas_call(
        _mm3_body,
        grid=(grid,),
        in_specs=[
            pl.BlockSpec((NC, _BR, 128), lambda i: (0, i, 0)),
            pl.BlockSpec((_BR, 128), lambda i: (i, 0)),
            pl.BlockSpec((_BR, 1), lambda i: (i, 0)),
            pl.BlockSpec((128,), lambda i: (0,)),
            pl.BlockSpec((128, 32), lambda i: (0, 0)),
            pl.BlockSpec((32,), lambda i: (0,)),
        ],
        out_specs=pl.BlockSpec((_BR, 32), lambda i: (i, 0)),
        out_shape=jax.ShapeDtypeStruct((N, 32), _f32),
    )(q, xws2, dinv, b2, Wfc, bfc)


_BD = 128


def _dec_body(zi_ref, zall_ref, y_ref):
    y = lax.dot_general(zi_ref[...], zall_ref[...],
                        (((1,), (1,)), ((), ())),
                        preferred_element_type=_f32)
    y = jax.nn.sigmoid(y)
    for k in range(_BD):
        y_ref[pl.ds(k * N, N)] = y[k]


def _dec(z):
    grid = (N + _BD - 1) // _BD
    return pl.pallas_call(
        _dec_body,
        grid=(grid,),
        in_specs=[
            pl.BlockSpec((_BD, 32), lambda i: (i, 0)),
            pl.BlockSpec((N, 32), lambda i: (0, 0)),
        ],
        out_specs=pl.BlockSpec((_BD * N,), lambda i: (i,)),
        out_shape=jax.ShapeDtypeStruct((N * N,), _f32),
    )(z, z)


def kernel(X, edge_list, W1, b1, W2, b2, Wfc, bfc):
    src = edge_list[0].astype(jnp.int32)
    dst = edge_list[1].astype(jnp.int32).reshape(NW * NSEG, SEG, K)

    ones128 = jnp.ones((K, 128), _f32)
    zeros128 = jnp.zeros((RPT_LAST, 128), _f32)

    W2p = jnp.pad(W2, ((0, 0), (0, 64)))
    b2p = jnp.pad(b2, (0, 64))
    Wfcp = jnp.pad(Wfc, ((0, 64), (0, 0)))

    degp = _make_deg_kernel()(dst, ones128, zeros128)
    xws1, dinv = _mm1(degp, X, W1)
    p = _make_agg_kernel(128)(xws1, src, dst, zeros128)
    xws2 = _mm2(p, xws1, dinv, b1, W2p)
    q = _make_agg_kernel(128)(xws2, src, dst, zeros128)
    z = _mm3(q, xws2, dinv, b2p, Wfcp, bfc)
    return _dec(z)


# speedup vs baseline: 1.0336x; 1.0336x over previous
import functools

import jax
import jax.numpy as jnp
from jax import lax
from jax.experimental import pallas as pl
from jax.experimental.pallas import tpu as pltpu
from jax.experimental.pallas import tpu_sc as plsc

N = 10000
E = 320000
NW = 32
NC = 2
NS = 16
EPW = E // NW
K = 80
NCHUNK = EPW // K
SEG = 25
NSEG = NCHUNK // SEG
NA = N
RPT = 624
RPT_LAST = N - (NS - 1) * RPT

_f32 = jnp.float32


def _rowpart(s, fn):
    @pl.when(s < NS - 1)
    def _():
        fn(pl.multiple_of(s * RPT, 8), RPT)

    @pl.when(s == NS - 1)
    def _():
        fn((NS - 1) * RPT, RPT_LAST)


def _sc_mesh():
    return plsc.VectorSubcoreMesh(core_axis_name="c", subcore_axis_name="s",
                                  num_cores=NC, num_subcores=NS)


@functools.lru_cache(maxsize=None)
def _make_deg_kernel():
    @functools.partial(
        pl.kernel,
        mesh=_sc_mesh(),
        out_type=jax.ShapeDtypeStruct((NC, N, 128), _f32),
        scratch_types=[
            pltpu.VMEM((SEG, K), jnp.int32),
            pltpu.VMEM((K, 128), _f32),
            pltpu.SemaphoreType.DMA,
            pltpu.VMEM_SHARED((NA, 128), _f32),
        ],
    )
    def deg_kernel(dst_hbm, ones_hbm, zeros_hbm, out_hbm, dst_v, ones_v, sem,
                   acc):
        c = lax.axis_index("c")
        s = lax.axis_index("s")
        wid = s * NC + c

        pltpu.sync_copy(ones_hbm, ones_v)
        _rowpart(s, lambda off, sz: pltpu.sync_copy(
            zeros_hbm.at[pl.ds(0, sz)], acc.at[pl.ds(off, sz)]))
        plsc.subcore_barrier()

        def seg_body(g, carry):
            pltpu.sync_copy(dst_hbm.at[wid * NSEG + g], dst_v)

            def fire(j, c2):
                pltpu.async_copy(ones_v, acc.at[dst_v.at[j]], sem, add=True)
                return c2

            lax.fori_loop(0, SEG, fire, 0)

            def drain(j, c2):
                pltpu.make_async_copy(ones_v, acc.at[dst_v.at[j]], sem).wait()
                return c2

            lax.fori_loop(0, SEG, drain, 0)
            return carry

        lax.fori_loop(0, NSEG, seg_body, 0)
        plsc.subcore_barrier()
        _rowpart(s, lambda off, sz: pltpu.sync_copy(
            acc.at[pl.ds(off, sz)], out_hbm.at[c].at[pl.ds(off, sz)]))

    return deg_kernel


@functools.lru_cache(maxsize=None)
def _make_agg_kernel(D):
    @functools.partial(
        pl.kernel,
        mesh=_sc_mesh(),
        out_type=jax.ShapeDtypeStruct((NC, N, D), _f32),
        scratch_types=[
            pltpu.VMEM((SEG, K), jnp.int32),
            pltpu.VMEM((SEG, K), jnp.int32),
            pltpu.VMEM((K, D), _f32),
            pltpu.VMEM((K, D), _f32),
            pltpu.SemaphoreType.DMA,
            pltpu.SemaphoreType.DMA,
            pltpu.VMEM_SHARED((NA, D), _f32),
        ],
    )
    def agg_kernel(xw_hbm, src_hbm, dst_hbm, zeros_hbm, out_hbm,
                   src_v, dst_v, rows_a, rows_b, sem_a, sem_b, acc):
        c = lax.axis_index("c")
        s = lax.axis_index("s")
        wid = s * NC + c

        _rowpart(s, lambda off, sz: pltpu.sync_copy(
            zeros_hbm.at[pl.ds(0, sz)], acc.at[pl.ds(off, sz)]))
        plsc.subcore_barrier()

        def sidx(j):
            return src_v.at[j]

        def seg_body(g, carry):
            pltpu.sync_copy(src_hbm.at[wid * NSEG + g], src_v)
            pltpu.sync_copy(dst_hbm.at[wid * NSEG + g], dst_v)
            pltpu.async_copy(xw_hbm.at[sidx(0)], rows_a, sem_a)

            def body(i, c2):
                j = 2 * i
                pltpu.async_copy(xw_hbm.at[sidx(j + 1)], rows_b, sem_b)
                pltpu.make_async_copy(
                    xw_hbm.at[sidx(j)], rows_a, sem_a).wait()
                pltpu.sync_copy(rows_a, acc.at[dst_v.at[j]], add=True)
                pltpu.async_copy(xw_hbm.at[sidx(j + 2)], rows_a, sem_a)
                pltpu.make_async_copy(
                    xw_hbm.at[sidx(j + 1)], rows_b, sem_b).wait()
                pltpu.sync_copy(rows_b, acc.at[dst_v.at[j + 1]], add=True)
                return c2

            lax.fori_loop(0, (SEG - 1) // 2, body, 0)
            pltpu.make_async_copy(xw_hbm.at[sidx(SEG - 1)], rows_a,
                                  sem_a).wait()
            pltpu.sync_copy(rows_a, acc.at[dst_v.at[SEG - 1]], add=True)
            return carry

        lax.fori_loop(0, NSEG, seg_body, 0)
        plsc.subcore_barrier()
        _rowpart(s, lambda off, sz: pltpu.sync_copy(
            acc.at[pl.ds(off, sz)], out_hbm.at[c].at[pl.ds(off, sz)]))

    return agg_kernel


_BR = 1000


def _mm1_body(deg_ref, x_ref, w_ref, xws_ref, dinv_ref):
    deg = deg_ref[0, :, 0:1] + deg_ref[1, :, 0:1] + 1.0
    dinv = lax.rsqrt(deg)
    xw = jnp.dot(x_ref[...], w_ref[...], preferred_element_type=_f32)
    xws_ref[...] = xw * dinv
    dinv_ref[...] = dinv


def _mm1(degp, X, W1):
    grid = N // _BR
    return pl.pallas_call(
        _mm1_body,
        grid=(grid,),
        in_specs=[
            pl.BlockSpec((NC, _BR, 128), lambda i: (0, i, 0)),
            pl.BlockSpec((_BR, 128), lambda i: (i, 0)),
            pl.BlockSpec((128, 128), lambda i: (0, 0)),
        ],
        out_specs=[
            pl.BlockSpec((_BR, 128), lambda i: (i, 0)),
            pl.BlockSpec((_BR, 1), lambda i: (i, 0)),
        ],
        out_shape=[
            jax.ShapeDtypeStruct((N, 128), _f32),
            jax.ShapeDtypeStruct((N, 1), _f32),
        ],
    )(degp, X, W1)


def _mm2_body(p_ref, xws_ref, dinv_ref, b1_ref, w2_ref, out_ref):
    dinv = dinv_ref[...]
    a = (p_ref[0] + p_ref[1] + xws_ref[...]) * dinv + b1_ref[...]
    a = jnp.maximum(a, 0.0)
    out_ref[...] = jnp.dot(a, w2_ref[...], preferred_element_type=_f32) * dinv


def _mm2(p, xws1, dinv, b1, W2):
    grid = N // _BR
    return pl.pallas_call(
        _mm2_body,
        grid=(grid,),
        in_specs=[
            pl.BlockSpec((NC, _BR, 128), lambda i: (0, i, 0)),
            pl.BlockSpec((_BR, 128), lambda i: (i, 0)),
            pl.BlockSpec((_BR, 1), lambda i: (i, 0)),
            pl.BlockSpec((128,), lambda i: (0,)),
            pl.BlockSpec((128, 128), lambda i: (0, 0)),
        ],
        out_specs=pl.BlockSpec((_BR, 128), lambda i: (i, 0)),
        out_shape=jax.ShapeDtypeStruct((N, 128), _f32),
    )(p, xws1, dinv, b1, W2)


def _mm3_body(q_ref, xws_ref, dinv_ref, b2_ref, wfc_ref, bfc_ref, z_ref):
    dinv = dinv_ref[...]
    h = (q_ref[0] + q_ref[1] + xws_ref[...]) * dinv + b2_ref[...]
    z = jnp.dot(h, wfc_ref[...], preferred_element_type=_f32) + bfc_ref[...]
    z_ref[...] = jnp.maximum(z, 0.0)


def _mm3(q, xws2, dinv, b2, Wfc, bfc):
    grid = N // _BR
    return pl.pallas_call(
        _mm3_body,
        grid=(grid,),
        in_specs=[
            pl.BlockSpec((NC, _BR, 128), lambda i: (0, i, 0)),
            pl.BlockSpec((_BR, 128), lambda i: (i, 0)),
            pl.BlockSpec((_BR, 1), lambda i: (i, 0)),
            pl.BlockSpec((128,), lambda i: (0,)),
            pl.BlockSpec((128, 32), lambda i: (0, 0)),
            pl.BlockSpec((32,), lambda i: (0,)),
        ],
        out_specs=pl.BlockSpec((_BR, 32), lambda i: (i, 0)),
        out_shape=jax.ShapeDtypeStruct((N, 32), _f32),
    )(q, xws2, dinv, b2, Wfc, bfc)


_BD = 256


def _dec_body(zi_ref, zall_ref, y_ref):
    y = lax.dot_general(zi_ref[...], zall_ref[...],
                        (((1,), (1,)), ((), ())),
                        preferred_element_type=_f32)
    y = jax.nn.sigmoid(y)
    for k in range(_BD):
        y_ref[pl.ds(k * N, N)] = y[k]


def _dec(z):
    grid = (N + _BD - 1) // _BD
    return pl.pallas_call(
        _dec_body,
        grid=(grid,),
        in_specs=[
            pl.BlockSpec((_BD, 32), lambda i: (i, 0)),
            pl.BlockSpec((N, 32), lambda i: (0, 0)),
        ],
        out_specs=pl.BlockSpec((_BD * N,), lambda i: (i,)),
        out_shape=jax.ShapeDtypeStruct((N * N,), _f32),
    )(z, z)


def kernel(X, edge_list, W1, b1, W2, b2, Wfc, bfc):
    src = edge_list[0].astype(jnp.int32).reshape(NW * NSEG, SEG, K)
    dst = edge_list[1].astype(jnp.int32).reshape(NW * NSEG, SEG, K)

    ones128 = jnp.ones((K, 128), _f32)
    zeros128 = jnp.zeros((RPT_LAST, 128), _f32)

    W2p = jnp.pad(W2, ((0, 0), (0, 64)))
    b2p = jnp.pad(b2, (0, 64))
    Wfcp = jnp.pad(Wfc, ((0, 64), (0, 0)))

    degp = _make_deg_kernel()(dst, ones128, zeros128)
    xws1, dinv = _mm1(degp, X, W1)
    p = _make_agg_kernel(128)(xws1, src, dst, zeros128)
    xws2 = _mm2(p, xws1, dinv, b1, W2p)
    q = _make_agg_kernel(128)(xws2, src, dst, zeros128)
    z = _mm3(q, xws2, dinv, b2p, Wfcp, bfc)
    return _dec(z)


# speedup vs baseline: 1.0595x; 1.0251x over previous
import functools

import jax
import jax.numpy as jnp
from jax import lax
from jax.experimental import pallas as pl
from jax.experimental.pallas import tpu as pltpu
from jax.experimental.pallas import tpu_sc as plsc

N = 10000
E = 320000
NW = 32
NC = 2
NS = 16
EPW = E // NW
K = 80
NCHUNK = EPW // K
SEG = 25
NSEG = NCHUNK // SEG
NA = N
RPT = 624
RPT_LAST = N - (NS - 1) * RPT

_f32 = jnp.float32


def _rowpart(s, fn):
    @pl.when(s < NS - 1)
    def _():
        fn(pl.multiple_of(s * RPT, 8), RPT)

    @pl.when(s == NS - 1)
    def _():
        fn((NS - 1) * RPT, RPT_LAST)


def _sc_mesh():
    return plsc.VectorSubcoreMesh(core_axis_name="c", subcore_axis_name="s",
                                  num_cores=NC, num_subcores=NS)


@functools.lru_cache(maxsize=None)
def _make_deg_kernel():
    @functools.partial(
        pl.kernel,
        mesh=_sc_mesh(),
        out_type=jax.ShapeDtypeStruct((NC, N, 128), _f32),
        scratch_types=[
            pltpu.VMEM((SEG, K), jnp.int32),
            pltpu.VMEM((K, 128), _f32),
            pltpu.SemaphoreType.DMA,
            pltpu.VMEM_SHARED((NA, 128), _f32),
        ],
    )
    def deg_kernel(dst_hbm, ones_hbm, zeros_hbm, out_hbm, dst_v, ones_v, sem,
                   acc):
        c = lax.axis_index("c")
        s = lax.axis_index("s")
        wid = s * NC + c

        pltpu.sync_copy(ones_hbm, ones_v)
        _rowpart(s, lambda off, sz: pltpu.sync_copy(
            zeros_hbm.at[pl.ds(0, sz)], acc.at[pl.ds(off, sz)]))
        plsc.subcore_barrier()

        def seg_body(g, carry):
            pltpu.sync_copy(dst_hbm.at[wid * NSEG + g], dst_v)

            def fire(j, c2):
                pltpu.async_copy(ones_v, acc.at[dst_v.at[j]], sem, add=True)
                return c2

            lax.fori_loop(0, SEG, fire, 0)

            def drain(j, c2):
                pltpu.make_async_copy(ones_v, acc.at[dst_v.at[j]], sem).wait()
                return c2

            lax.fori_loop(0, SEG, drain, 0)
            return carry

        lax.fori_loop(0, NSEG, seg_body, 0)
        plsc.subcore_barrier()
        _rowpart(s, lambda off, sz: pltpu.sync_copy(
            acc.at[pl.ds(off, sz)], out_hbm.at[c].at[pl.ds(off, sz)]))

    return deg_kernel


@functools.lru_cache(maxsize=None)
def _make_agg_kernel(D):
    @functools.partial(
        pl.kernel,
        mesh=_sc_mesh(),
        out_type=jax.ShapeDtypeStruct((NC, N, D), _f32),
        scratch_types=[
            pltpu.VMEM((SEG, K), jnp.int32),
            pltpu.VMEM((SEG, K), jnp.int32),
            pltpu.VMEM((K, D), _f32),
            pltpu.VMEM((K, D), _f32),
            pltpu.SemaphoreType.DMA,
            pltpu.SemaphoreType.DMA,
            pltpu.VMEM_SHARED((NA, D), _f32),
        ],
    )
    def agg_kernel(xw_hbm, src_hbm, dst_hbm, zeros_hbm, out_hbm,
                   src_v, dst_v, rows_a, rows_b, sem_a, sem_b, acc):
        c = lax.axis_index("c")
        s = lax.axis_index("s")
        wid = s * NC + c

        _rowpart(s, lambda off, sz: pltpu.sync_copy(
            zeros_hbm.at[pl.ds(0, sz)], acc.at[pl.ds(off, sz)]))
        plsc.subcore_barrier()

        def sidx(j):
            return src_v.at[j]

        def seg_body(g, carry):
            pltpu.sync_copy(src_hbm.at[wid * NSEG + g], src_v)
            pltpu.sync_copy(dst_hbm.at[wid * NSEG + g], dst_v)
            pltpu.async_copy(xw_hbm.at[sidx(0)], rows_a, sem_a)

            def body(i, c2):
                j = 2 * i
                pltpu.async_copy(xw_hbm.at[sidx(j + 1)], rows_b, sem_b)
                pltpu.make_async_copy(
                    xw_hbm.at[sidx(j)], rows_a, sem_a).wait()
                pltpu.sync_copy(rows_a, acc.at[dst_v.at[j]], add=True)
                pltpu.async_copy(xw_hbm.at[sidx(j + 2)], rows_a, sem_a)
                pltpu.make_async_copy(
                    xw_hbm.at[sidx(j + 1)], rows_b, sem_b).wait()
                pltpu.sync_copy(rows_b, acc.at[dst_v.at[j + 1]], add=True)
                return c2

            lax.fori_loop(0, (SEG - 1) // 2, body, 0)
            pltpu.make_async_copy(xw_hbm.at[sidx(SEG - 1)], rows_a,
                                  sem_a).wait()
            pltpu.sync_copy(rows_a, acc.at[dst_v.at[SEG - 1]], add=True)
            return carry

        lax.fori_loop(0, NSEG, seg_body, 0)
        plsc.subcore_barrier()
        _rowpart(s, lambda off, sz: pltpu.sync_copy(
            acc.at[pl.ds(off, sz)], out_hbm.at[c].at[pl.ds(off, sz)]))

    return agg_kernel


_BR = 1000


def _mm1_body(deg_ref, x_ref, w_ref, xws_ref, dinv_ref):
    deg = deg_ref[0, :, 0:1] + deg_ref[1, :, 0:1] + 1.0
    dinv = lax.rsqrt(deg)
    xw = jnp.dot(x_ref[...], w_ref[...], preferred_element_type=_f32)
    xws_ref[...] = xw * dinv
    dinv_ref[...] = dinv


def _mm1(degp, X, W1):
    grid = N // _BR
    return pl.pallas_call(
        _mm1_body,
        grid=(grid,),
        in_specs=[
            pl.BlockSpec((NC, _BR, 128), lambda i: (0, i, 0)),
            pl.BlockSpec((_BR, 128), lambda i: (i, 0)),
            pl.BlockSpec((128, 128), lambda i: (0, 0)),
        ],
        out_specs=[
            pl.BlockSpec((_BR, 128), lambda i: (i, 0)),
            pl.BlockSpec((_BR, 1), lambda i: (i, 0)),
        ],
        out_shape=[
            jax.ShapeDtypeStruct((N, 128), _f32),
            jax.ShapeDtypeStruct((N, 1), _f32),
        ],
    )(degp, X, W1)


def _mm2_body(p_ref, xws_ref, dinv_ref, b1_ref, w2_ref, out_ref):
    dinv = dinv_ref[...]
    a = (p_ref[0] + p_ref[1] + xws_ref[...]) * dinv + b1_ref[...]
    a = jnp.maximum(a, 0.0)
    out_ref[...] = jnp.dot(a, w2_ref[...], preferred_element_type=_f32) * dinv


def _mm2(p, xws1, dinv, b1, W2):
    grid = N // _BR
    return pl.pallas_call(
        _mm2_body,
        grid=(grid,),
        in_specs=[
            pl.BlockSpec((NC, _BR, 128), lambda i: (0, i, 0)),
            pl.BlockSpec((_BR, 128), lambda i: (i, 0)),
            pl.BlockSpec((_BR, 1), lambda i: (i, 0)),
            pl.BlockSpec((128,), lambda i: (0,)),
            pl.BlockSpec((128, 128), lambda i: (0, 0)),
        ],
        out_specs=pl.BlockSpec((_BR, 128), lambda i: (i, 0)),
        out_shape=jax.ShapeDtypeStruct((N, 128), _f32),
    )(p, xws1, dinv, b1, W2)


def _mm3_body(q_ref, xws_ref, dinv_ref, b2_ref, wfc_ref, bfc_ref, z_ref):
    dinv = dinv_ref[...]
    h = (q_ref[0] + q_ref[1] + xws_ref[...]) * dinv + b2_ref[...]
    z = jnp.dot(h, wfc_ref[...], preferred_element_type=_f32) + bfc_ref[...]
    z_ref[...] = jnp.maximum(z, 0.0)


def _mm3(q, xws2, dinv, b2, Wfc, bfc):
    grid = N // _BR
    return pl.pallas_call(
        _mm3_body,
        grid=(grid,),
        in_specs=[
            pl.BlockSpec((NC, _BR, 128), lambda i: (0, i, 0)),
            pl.BlockSpec((_BR, 128), lambda i: (i, 0)),
            pl.BlockSpec((_BR, 1), lambda i: (i, 0)),
            pl.BlockSpec((128,), lambda i: (0,)),
            pl.BlockSpec((128, 32), lambda i: (0, 0)),
            pl.BlockSpec((32,), lambda i: (0,)),
        ],
        out_specs=pl.BlockSpec((_BR, 32), lambda i: (i, 0)),
        out_shape=jax.ShapeDtypeStruct((N, 32), _f32),
    )(q, xws2, dinv, b2, Wfc, bfc)


_BD = 512


def _dec_body(zi_ref, zall_ref, y_ref):
    y = lax.dot_general(zi_ref[...], zall_ref[...],
                        (((1,), (1,)), ((), ())),
                        preferred_element_type=_f32)
    y = jax.nn.sigmoid(y)
    for k in range(_BD):
        y_ref[pl.ds(k * N, N)] = y[k]


def _dec(z):
    grid = (N + _BD - 1) // _BD
    return pl.pallas_call(
        _dec_body,
        grid=(grid,),
        in_specs=[
            pl.BlockSpec((_BD, 32), lambda i: (i, 0)),
            pl.BlockSpec((N, 32), lambda i: (0, 0)),
        ],
        out_specs=pl.BlockSpec((_BD * N,), lambda i: (i,)),
        out_shape=jax.ShapeDtypeStruct((N * N,), _f32),
    )(z, z)


def kernel(X, edge_list, W1, b1, W2, b2, Wfc, bfc):
    src = edge_list[0].astype(jnp.int32).reshape(NW * NSEG, SEG, K)
    dst = edge_list[1].astype(jnp.int32).reshape(NW * NSEG, SEG, K)

    ones128 = jnp.ones((K, 128), _f32)
    zeros128 = jnp.zeros((RPT_LAST, 128), _f32)

    W2p = jnp.pad(W2, ((0, 0), (0, 64)))
    b2p = jnp.pad(b2, (0, 64))
    Wfcp = jnp.pad(Wfc, ((0, 64), (0, 0)))

    degp = _make_deg_kernel()(dst, ones128, zeros128)
    xws1, dinv = _mm1(degp, X, W1)
    p = _make_agg_kernel(128)(xws1, src, dst, zeros128)
    xws2 = _mm2(p, xws1, dinv, b1, W2p)
    q = _make_agg_kernel(128)(xws2, src, dst, zeros128)
    z = _mm3(q, xws2, dinv, b2p, Wfcp, bfc)
    return _dec(z)
